# baseline (speedup 1.0000x reference)
import functools

import jax
import jax.numpy as jnp
from jax import lax
from jax.experimental import pallas as pl
from jax.experimental.pallas import tpu as pltpu
from jax.experimental.pallas import tpu_sc as plsc

_NC = 2
_NS = 16
_NW = _NC * _NS
_TS = 8
_NXB = 2
_NTB = 2
_S_SC = 256
_BS = 2048


def _sc_part(inputs, table):
    B, S, D = inputs.shape
    rows_w = _S_SC // _NW
    tiles_w = rows_w // _TS

    mesh = plsc.VectorSubcoreMesh(core_axis_name="c", subcore_axis_name="s")

    scratch = (
        [pltpu.VMEM((B, _TS, D), jnp.float32) for _ in range(_NXB)]
        + [pltpu.VMEM((_TS, D), jnp.float32) for _ in range(_NTB)]
        + [pltpu.SemaphoreType.DMA] * (2 * _NXB + _NTB)
    )

    @functools.partial(
        pl.kernel,
        out_type=jax.ShapeDtypeStruct((B, _S_SC, D), jnp.float32),
        mesh=mesh,
        scratch_types=scratch,
    )
    def sc_add(x_hbm, t_hbm, o_hbm, *bufs):
        xb = bufs[:_NXB]
        tb = bufs[_NXB:_NXB + _NTB]
        xin_sem = bufs[_NXB + _NTB:2 * _NXB + _NTB]
        xout_sem = bufs[2 * _NXB + _NTB:3 * _NXB + _NTB]
        tin_sem = bufs[3 * _NXB + _NTB:]

        wid = lax.axis_index("s") * _NC + lax.axis_index("c")
        base = wid * rows_w

        def start_in(t):
            p = t % _NXB
            return pltpu.async_copy(
                x_hbm.at[:, pl.ds(base + t * _TS, _TS), :], xb[p],
                xin_sem[p])

        def start_tab(t):
            q = t % _NTB
            return pltpu.async_copy(
                t_hbm.at[pl.ds(base + t * _TS, _TS), :], tb[q], tin_sem[q])

        in_d = {}
        out_d = {}
        tab_d = {}
        for t in range(min(_NTB, tiles_w)):
            tab_d[t] = start_tab(t)
        in_d[0] = start_in(0)

        for t in range(tiles_w):
            p = t % _NXB

            v = t + 1
            if v < tiles_w:
                if v - _NXB >= 0:
                    out_d[v - _NXB].wait()
                in_d[v] = start_in(v)

            tab_d[t].wait()
            in_d[t].wait()

            tbq = tb[t % _NTB]
            xbp = xb[p]

            for r in range(_TS):
                @plsc.parallel_loop(0, D, step=16, unroll=8)
                def _add(i):
                    for b in range(B):
                        xbp[b, r, pl.ds(i, 16)] = (
                            xbp[b, r, pl.ds(i, 16)] + tbq[r, pl.ds(i, 16)])

            out_d[t] = pltpu.async_copy(
                xbp, o_hbm.at[:, pl.ds(base + t * _TS, _TS), :],
                xout_sem[p])

            if t + _NTB < tiles_w:
                tab_d[t + _NTB] = start_tab(t + _NTB)

        for t in range(max(0, tiles_w - _NXB), tiles_w):
            out_d[t].wait()

    return sc_add(inputs, table)


def _tc_body(x_ref, t_ref, sc_ref, o_ref, stage, sem):
    i = pl.program_id(0)
    j = pl.program_id(1)
    nblk = pl.num_programs(0)

    @pl.when((i == 0) & (j == 0))
    def _prefetch_sc_rows():
        pltpu.async_copy(sc_ref, stage, sem)

    @pl.when((i == nblk - 1) & (j == 0))
    def _wait_sc_rows():
        pltpu.make_async_copy(sc_ref, stage, sem).wait()

    o_ref[...] = x_ref[...] + t_ref[...]

    @pl.when(i == nblk - 1)
    def _merge_block():
        o_ref[0, :_S_SC, :] = stage[j]


def _tc_part(inputs, table, sc_out):
    B, S, D = inputs.shape
    nblk = S // _BS
    blk = lambda i: (i + 1) % nblk
    grid = (nblk, B)
    return pl.pallas_call(
        _tc_body,
        grid=grid,
        in_specs=[
            pl.BlockSpec((1, _BS, D), lambda i, j: (j, blk(i), 0)),
            pl.BlockSpec((_BS, D), lambda i, j: (blk(i), 0)),
            pl.BlockSpec(memory_space=pltpu.HBM),
        ],
        out_specs=pl.BlockSpec((1, _BS, D), lambda i, j: (j, blk(i), 0)),
        out_shape=jax.ShapeDtypeStruct((B, S, D), inputs.dtype),
        scratch_shapes=[
            pltpu.VMEM((B, _S_SC, D), inputs.dtype),
            pltpu.SemaphoreType.DMA,
        ],
    )(inputs, table, sc_out)


def kernel(inputs, table):
    sc_out = _sc_part(inputs, table)
    return _tc_part(inputs, table, sc_out)


# speedup vs baseline: 1.0354x; 1.0354x over previous
import functools

import jax
import jax.numpy as jnp
from jax import lax
from jax.experimental import pallas as pl
from jax.experimental.pallas import tpu as pltpu
from jax.experimental.pallas import tpu_sc as plsc

_NC = 2
_NS = 16
_NW = _NC * _NS
_TS = 4
_NXB = 2
_NTB = 2
_S_SC = 128
_BS = 2048


def _sc_part(inputs, table):
    B, S, D = inputs.shape
    rows_w = _S_SC // _NW
    tiles_w = rows_w // _TS

    mesh = plsc.VectorSubcoreMesh(core_axis_name="c", subcore_axis_name="s")

    scratch = (
        [pltpu.VMEM((B, _TS, D), jnp.float32) for _ in range(_NXB)]
        + [pltpu.VMEM((_TS, D), jnp.float32) for _ in range(_NTB)]
        + [pltpu.SemaphoreType.DMA] * (2 * _NXB + _NTB)
    )

    @functools.partial(
        pl.kernel,
        out_type=jax.ShapeDtypeStruct((B, _S_SC, D), jnp.float32),
        mesh=mesh,
        scratch_types=scratch,
    )
    def sc_add(x_hbm, t_hbm, o_hbm, *bufs):
        xb = bufs[:_NXB]
        tb = bufs[_NXB:_NXB + _NTB]
        xin_sem = bufs[_NXB + _NTB:2 * _NXB + _NTB]
        xout_sem = bufs[2 * _NXB + _NTB:3 * _NXB + _NTB]
        tin_sem = bufs[3 * _NXB + _NTB:]

        wid = lax.axis_index("s") * _NC + lax.axis_index("c")
        base = wid * rows_w

        def start_in(t):
            p = t % _NXB
            return pltpu.async_copy(
                x_hbm.at[:, pl.ds(base + t * _TS, _TS), :], xb[p],
                xin_sem[p])

        def start_tab(t):
            q = t % _NTB
            return pltpu.async_copy(
                t_hbm.at[pl.ds(base + t * _TS, _TS), :], tb[q], tin_sem[q])

        in_d = {}
        out_d = {}
        tab_d = {}
        for t in range(min(_NTB, tiles_w)):
            tab_d[t] = start_tab(t)
        in_d[0] = start_in(0)

        for t in range(tiles_w):
            p = t % _NXB

            v = t + 1
            if v < tiles_w:
                if v - _NXB >= 0:
                    out_d[v - _NXB].wait()
                in_d[v] = start_in(v)

            tab_d[t].wait()
            in_d[t].wait()

            tbq = tb[t % _NTB]
            xbp = xb[p]

            for r in range(_TS):
                @plsc.parallel_loop(0, D, step=16, unroll=8)
                def _add(i):
                    for b in range(B):
                        xbp[b, r, pl.ds(i, 16)] = (
                            xbp[b, r, pl.ds(i, 16)] + tbq[r, pl.ds(i, 16)])

            out_d[t] = pltpu.async_copy(
                xbp, o_hbm.at[:, pl.ds(base + t * _TS, _TS), :],
                xout_sem[p])

            if t + _NTB < tiles_w:
                tab_d[t + _NTB] = start_tab(t + _NTB)

        for t in range(max(0, tiles_w - _NXB), tiles_w):
            out_d[t].wait()

    return sc_add(inputs, table)


def _tc_body(x_ref, t_ref, sc_ref, o_ref, stage, sem):
    i = pl.program_id(0)
    j = pl.program_id(1)
    nblk = pl.num_programs(0)

    @pl.when((i == 0) & (j == 0))
    def _prefetch_sc_rows():
        pltpu.async_copy(sc_ref, stage, sem)

    @pl.when((i == nblk - 1) & (j == 0))
    def _wait_sc_rows():
        pltpu.make_async_copy(sc_ref, stage, sem).wait()

    o_ref[...] = x_ref[...] + t_ref[...]

    @pl.when(i == nblk - 1)
    def _merge_block():
        o_ref[0, :_S_SC, :] = stage[j]


def _tc_part(inputs, table, sc_out):
    B, S, D = inputs.shape
    nblk = S // _BS
    blk = lambda i: (i + 1) % nblk
    grid = (nblk, B)
    return pl.pallas_call(
        _tc_body,
        grid=grid,
        in_specs=[
            pl.BlockSpec((1, _BS, D), lambda i, j: (j, blk(i), 0)),
            pl.BlockSpec((_BS, D), lambda i, j: (blk(i), 0)),
            pl.BlockSpec(memory_space=pltpu.HBM),
        ],
        out_specs=pl.BlockSpec((1, _BS, D), lambda i, j: (j, blk(i), 0)),
        out_shape=jax.ShapeDtypeStruct((B, S, D), inputs.dtype),
        scratch_shapes=[
            pltpu.VMEM((B, _S_SC, D), inputs.dtype),
            pltpu.SemaphoreType.DMA,
        ],
    )(inputs, table, sc_out)


def kernel(inputs, table):
    sc_out = _sc_part(inputs, table)
    return _tc_part(inputs, table, sc_out)
